# baseline (speedup 1.0000x reference)
import functools

import jax
import jax.numpy as jnp
from jax import lax
from jax.experimental import pallas as pl
from jax.experimental.pallas import tpu as pltpu
from jax.experimental.pallas import tpu_sc as plsc

T = 6
NC = 2
NS = 16
CH = 125
NB = 2


def _edge_table_body(emb_ref, w_ref, b_ref, src_ref, et_ref, out_ref, g_ref):
    out_ref[...] = (
        jnp.dot(emb_ref[...], w_ref[...], preferred_element_type=jnp.float32)
        + b_ref[...]
    )
    g_ref[...] = src_ref[...] * T + et_ref[...]


def _edge_table(emb, w_edge, b_edge, src2d, et2d):
    n, h = emb.shape
    th = w_edge.shape[1]
    er, ec = src2d.shape
    blk = 1000
    grid = n // blk
    eblk = er // grid
    espec = pl.BlockSpec((eblk, ec), lambda i: (i, 0))
    return pl.pallas_call(
        _edge_table_body,
        grid=(grid,),
        in_specs=[
            pl.BlockSpec((blk, h), lambda i: (i, 0)),
            pl.BlockSpec((h, th), lambda i: (0, 0)),
            pl.BlockSpec((1, th), lambda i: (0, 0)),
            espec, espec,
        ],
        out_specs=[pl.BlockSpec((blk, th), lambda i: (i, 0)), espec],
        out_shape=[jax.ShapeDtypeStruct((n, th), jnp.float32),
                   jax.ShapeDtypeStruct((er, ec), jnp.int32)],
    )(emb, w_edge, b_edge.reshape(1, th), src2d, et2d)


SB = 16


def _make_sc_scatter(npad, h, e):
    ew = e // (NC * NS)
    nch = ew // CH
    nsb = nch // SB
    nq = SB // NB
    rows_per_sub = npad // NS
    mesh = plsc.VectorSubcoreMesh(
        core_axis_name="c", subcore_axis_name="s", num_cores=NC, num_subcores=NS
    )

    @functools.partial(
        pl.kernel,
        out_type=jax.ShapeDtypeStruct((NC, npad, h), jnp.float32),
        mesh=mesh,
        scratch_types=[
            pltpu.VMEM((SB, CH), jnp.int32),
            pltpu.VMEM((SB, CH), jnp.int32),
            pltpu.VMEM((NB, CH, h), jnp.float32),
            pltpu.VMEM_SHARED((npad, h), jnp.float32),
            [pltpu.SemaphoreType.DMA] * NB,
            [pltpu.SemaphoreType.DMA] * NB,
        ],
    )
    def sc_scatter(xall, g4, d4, out, gidx, didx, rows, acc, gsem, ssem):
        c = lax.axis_index("c")
        s = lax.axis_index("s")
        w = c * NS + s

        def zbody(i, _):
            for k in range(h // 16):
                rows[0, i, pl.ds(k * 16, 16)] = jnp.zeros((16,), jnp.float32)
            return 0

        lax.fori_loop(0, CH, zbody, 0)
        nfull, rem = rows_per_sub // CH, rows_per_sub % CH
        for q in range(nfull):
            pltpu.sync_copy(
                rows.at[0], acc.at[pl.ds(s * rows_per_sub + q * CH, CH)])
        if rem:
            pltpu.sync_copy(
                rows.at[0, :rem], acc.at[pl.ds(s * rows_per_sub + nfull * CH, rem)])
        plsc.subcore_barrier()

        def sb_body(sb, _):
            pltpu.sync_copy(g4.at[w, sb], gidx)
            pltpu.sync_copy(d4.at[w, sb], didx)

            for b in range(NB):
                pltpu.async_copy(xall.at[gidx.at[b]], rows.at[b], gsem[b])

            def quad_body(q, _):
                j = q * NB
                for b in range(NB):
                    pltpu.make_async_copy(
                        xall.at[gidx.at[j + b]], rows.at[b], gsem[b]).wait()
                    pltpu.sync_copy(rows.at[b], acc.at[didx.at[j + b]], add=True)
                    pltpu.async_copy(
                        xall.at[gidx.at[j + NB + b]], rows.at[b], gsem[b])
                return 0

            lax.fori_loop(0, nq - 1, quad_body, 0)

            j = SB - NB
            for b in range(NB):
                pltpu.make_async_copy(
                    xall.at[gidx.at[j + b]], rows.at[b], gsem[b]).wait()
                pltpu.sync_copy(rows.at[b], acc.at[didx.at[j + b]], add=True)
            return 0

        lax.fori_loop(0, nsb, sb_body, 0)
        plsc.subcore_barrier()

        pltpu.sync_copy(
            acc.at[pl.ds(s * rows_per_sub, rows_per_sub)],
            out.at[c, pl.ds(s * rows_per_sub, rows_per_sub)],
        )

    return sc_scatter


def _gru_body(i_ref, p_ref, wir_ref, bir_ref, wiz_ref, biz_ref,
              win_ref, bin_ref, whr_ref, whz_ref, whn_ref, bhn_ref, out_ref):
    x = i_ref[...]
    h = p_ref[0] + p_ref[1]

    def mm(a, w_ref):
        return jnp.dot(a, w_ref[...], preferred_element_type=jnp.float32)

    r = jax.nn.sigmoid(mm(x, wir_ref) + bir_ref[...] + mm(h, whr_ref))
    z = jax.nn.sigmoid(mm(x, wiz_ref) + biz_ref[...] + mm(h, whz_ref))
    nn = jnp.tanh(mm(x, win_ref) + bin_ref[...] + r * (mm(h, whn_ref) + bhn_ref[...]))
    out_ref[...] = (1.0 - z) * nn + z * h


def _gru(emb, partials, w_ir, b_ir, w_iz, b_iz, w_in, b_in, w_hr, w_hz,
         w_hn, b_hn):
    n, h = emb.shape
    blk = 1000
    grid = n // blk
    row_spec = pl.BlockSpec((blk, h), lambda i: (i, 0))
    p_spec = pl.BlockSpec((2, blk, h), lambda i: (0, i, 0))
    w_spec = pl.BlockSpec((h, h), lambda i: (0, 0))
    b_spec = pl.BlockSpec((1, h), lambda i: (0, 0))
    return pl.pallas_call(
        _gru_body,
        grid=(grid,),
        in_specs=[row_spec, p_spec,
                  w_spec, b_spec, w_spec, b_spec, w_spec, b_spec,
                  w_spec, w_spec, w_spec, b_spec],
        out_specs=row_spec,
        out_shape=jax.ShapeDtypeStruct((n, h), jnp.float32),
    )(emb, partials,
      w_ir, b_ir.reshape(1, h), w_iz, b_iz.reshape(1, h),
      w_in, b_in.reshape(1, h),
      w_hr, w_hz, w_hn, b_hn.reshape(1, h))


def kernel(statement_embeddings, source_indices, dest_indices, edge_types,
           num_nodes, hidden_size, config, W_edge, b_edge, W_ir, b_ir,
           W_iz, b_iz, W_in, b_in, W_hr, W_hz, W_hn, b_hn):
    n, h = statement_embeddings.shape
    e = source_indices.shape[0]

    npad = ((n + 8 * NS - 1) // (8 * NS)) * (8 * NS)
    nw = NC * NS
    nsb = e // (nw * SB * CH)
    x_all, g2d = _edge_table(statement_embeddings, W_edge, b_edge,
                             source_indices.reshape(e // CH, CH),
                             edge_types.reshape(e // CH, CH))
    x_all = x_all.reshape(n * T, h)
    g4 = g2d.reshape(nw, nsb, SB, CH)
    dest4 = dest_indices.reshape(nw, nsb, SB, CH)
    sc = _make_sc_scatter(npad, h, e)
    partials = sc(x_all, g4, dest4)

    return _gru(statement_embeddings, partials,
                W_ir, b_ir, W_iz, b_iz, W_in, b_in, W_hr, W_hz, W_hn, b_hn)


# speedup vs baseline: 1.0467x; 1.0467x over previous
import functools

import jax
import jax.numpy as jnp
from jax import lax
from jax.experimental import pallas as pl
from jax.experimental.pallas import tpu as pltpu
from jax.experimental.pallas import tpu_sc as plsc

T = 6
NC = 2
NS = 16
CH = 125
NB = 2


def _edge_table_body(emb_ref, w_ref, b_ref, out_ref):
    out_ref[...] = (
        jnp.dot(emb_ref[...], w_ref[...], preferred_element_type=jnp.float32)
        + b_ref[...]
    )


def _edge_table(emb, w_edge, b_edge):
    n, h = emb.shape
    th = w_edge.shape[1]
    blk = 2000
    grid = n // blk
    return pl.pallas_call(
        _edge_table_body,
        grid=(grid,),
        in_specs=[
            pl.BlockSpec((blk, h), lambda i: (i, 0)),
            pl.BlockSpec((h, th), lambda i: (0, 0)),
            pl.BlockSpec((1, th), lambda i: (0, 0)),
        ],
        out_specs=pl.BlockSpec((blk, th), lambda i: (i, 0)),
        out_shape=jax.ShapeDtypeStruct((n, th), jnp.float32),
    )(emb, w_edge, b_edge.reshape(1, th))


SB = 16


def _make_sc_scatter(npad, h, e):
    ew = e // (NC * NS)
    nch = ew // CH
    nsb = nch // SB
    nq = SB // NB
    rows_per_sub = npad // NS
    mesh = plsc.VectorSubcoreMesh(
        core_axis_name="c", subcore_axis_name="s", num_cores=NC, num_subcores=NS
    )

    @functools.partial(
        pl.kernel,
        out_type=jax.ShapeDtypeStruct((NC, npad, h), jnp.float32),
        mesh=mesh,
        scratch_types=[
            pltpu.VMEM((SB, CH), jnp.int32),
            pltpu.VMEM((SB, CH), jnp.int32),
            pltpu.VMEM((NB, CH, h), jnp.float32),
            pltpu.VMEM_SHARED((npad, h), jnp.float32),
            [pltpu.SemaphoreType.DMA] * NB,
            [pltpu.SemaphoreType.DMA] * NB,
        ],
    )
    def sc_scatter(xall, g4, d4, out, gidx, didx, rows, acc, gsem, ssem):
        c = lax.axis_index("c")
        s = lax.axis_index("s")
        w = c * NS + s

        def zbody(i, _):
            for k in range(h // 16):
                rows[0, i, pl.ds(k * 16, 16)] = jnp.zeros((16,), jnp.float32)
            return 0

        lax.fori_loop(0, CH, zbody, 0)
        nfull, rem = rows_per_sub // CH, rows_per_sub % CH
        for q in range(nfull):
            pltpu.sync_copy(
                rows.at[0], acc.at[pl.ds(s * rows_per_sub + q * CH, CH)])
        if rem:
            pltpu.sync_copy(
                rows.at[0, :rem], acc.at[pl.ds(s * rows_per_sub + nfull * CH, rem)])
        plsc.subcore_barrier()

        def sb_body(sb, _):
            pltpu.sync_copy(g4.at[w, sb], gidx)
            pltpu.sync_copy(d4.at[w, sb], didx)

            for b in range(NB):
                pltpu.async_copy(xall.at[gidx.at[b]], rows.at[b], gsem[b])

            def quad_body(q, _):
                j = q * NB
                for b in range(NB):
                    pltpu.make_async_copy(
                        xall.at[gidx.at[j + b]], rows.at[b], gsem[b]).wait()
                    pltpu.sync_copy(rows.at[b], acc.at[didx.at[j + b]], add=True)
                    pltpu.async_copy(
                        xall.at[gidx.at[j + NB + b]], rows.at[b], gsem[b])
                return 0

            lax.fori_loop(0, nq - 1, quad_body, 0)

            j = SB - NB
            for b in range(NB):
                pltpu.make_async_copy(
                    xall.at[gidx.at[j + b]], rows.at[b], gsem[b]).wait()
                pltpu.sync_copy(rows.at[b], acc.at[didx.at[j + b]], add=True)
            return 0

        lax.fori_loop(0, nsb, sb_body, 0)
        plsc.subcore_barrier()

        pltpu.sync_copy(
            acc.at[pl.ds(s * rows_per_sub, rows_per_sub)],
            out.at[c, pl.ds(s * rows_per_sub, rows_per_sub)],
        )

    return sc_scatter


def _gru_body(i_ref, p_ref, wir_ref, bir_ref, wiz_ref, biz_ref,
              win_ref, bin_ref, whr_ref, whz_ref, whn_ref, bhn_ref, out_ref):
    x = i_ref[...]
    h = p_ref[0] + p_ref[1]

    def mm(a, w_ref):
        return jnp.dot(a, w_ref[...], preferred_element_type=jnp.float32)

    r = jax.nn.sigmoid(mm(x, wir_ref) + bir_ref[...] + mm(h, whr_ref))
    z = jax.nn.sigmoid(mm(x, wiz_ref) + biz_ref[...] + mm(h, whz_ref))
    nn = jnp.tanh(mm(x, win_ref) + bin_ref[...] + r * (mm(h, whn_ref) + bhn_ref[...]))
    out_ref[...] = (1.0 - z) * nn + z * h


def _gru(emb, partials, w_ir, b_ir, w_iz, b_iz, w_in, b_in, w_hr, w_hz,
         w_hn, b_hn):
    n, h = emb.shape
    blk = 2000
    grid = n // blk
    row_spec = pl.BlockSpec((blk, h), lambda i: (i, 0))
    p_spec = pl.BlockSpec((2, blk, h), lambda i: (0, i, 0))
    w_spec = pl.BlockSpec((h, h), lambda i: (0, 0))
    b_spec = pl.BlockSpec((1, h), lambda i: (0, 0))
    return pl.pallas_call(
        _gru_body,
        grid=(grid,),
        in_specs=[row_spec, p_spec,
                  w_spec, b_spec, w_spec, b_spec, w_spec, b_spec,
                  w_spec, w_spec, w_spec, b_spec],
        out_specs=row_spec,
        out_shape=jax.ShapeDtypeStruct((n, h), jnp.float32),
    )(emb, partials,
      w_ir, b_ir.reshape(1, h), w_iz, b_iz.reshape(1, h),
      w_in, b_in.reshape(1, h),
      w_hr, w_hz, w_hn, b_hn.reshape(1, h))


def kernel(statement_embeddings, source_indices, dest_indices, edge_types,
           num_nodes, hidden_size, config, W_edge, b_edge, W_ir, b_ir,
           W_iz, b_iz, W_in, b_in, W_hr, W_hz, W_hn, b_hn):
    n, h = statement_embeddings.shape
    e = source_indices.shape[0]

    npad = ((n + 8 * NS - 1) // (8 * NS)) * (8 * NS)
    nw = NC * NS
    nsb = e // (nw * SB * CH)
    x_all = _edge_table(statement_embeddings, W_edge, b_edge)
    x_all = x_all.reshape(n * T, h)
    g4 = (source_indices * T + edge_types).reshape(nw, nsb, SB, CH)
    dest4 = dest_indices.reshape(nw, nsb, SB, CH)
    sc = _make_sc_scatter(npad, h, e)
    partials = sc(x_all, g4, dest4)

    return _gru(statement_embeddings, partials,
                W_ir, b_ir, W_iz, b_iz, W_in, b_in, W_hr, W_hz, W_hn, b_hn)


# speedup vs baseline: 1.0729x; 1.0251x over previous
import functools

import jax
import jax.numpy as jnp
from jax import lax
from jax.experimental import pallas as pl
from jax.experimental.pallas import tpu as pltpu
from jax.experimental.pallas import tpu_sc as plsc

T = 6
NC = 2
NS = 16
CH = 100
NB = 2


def _edge_table_body(emb_ref, w_ref, b_ref, out_ref):
    out_ref[...] = (
        jnp.dot(emb_ref[...], w_ref[...], preferred_element_type=jnp.float32)
        + b_ref[...]
    )


def _edge_table(emb, w_edge, b_edge):
    n, h = emb.shape
    th = w_edge.shape[1]
    blk = 2000
    grid = n // blk
    return pl.pallas_call(
        _edge_table_body,
        grid=(grid,),
        in_specs=[
            pl.BlockSpec((blk, h), lambda i: (i, 0)),
            pl.BlockSpec((h, th), lambda i: (0, 0)),
            pl.BlockSpec((1, th), lambda i: (0, 0)),
        ],
        out_specs=pl.BlockSpec((blk, th), lambda i: (i, 0)),
        out_shape=jax.ShapeDtypeStruct((n, th), jnp.float32),
    )(emb, w_edge, b_edge.reshape(1, th))


SB = 20


def _make_sc_scatter(npad, h, e):
    ew = e // (NC * NS)
    nch = ew // CH
    nsb = nch // SB
    rows_per_sub = npad // NS
    mesh = plsc.VectorSubcoreMesh(
        core_axis_name="c", subcore_axis_name="s", num_cores=NC, num_subcores=NS
    )

    @functools.partial(
        pl.kernel,
        out_type=jax.ShapeDtypeStruct((NC, npad, h), jnp.float32),
        mesh=mesh,
        scratch_types=[
            pltpu.VMEM((2, SB, CH), jnp.int32),
            pltpu.VMEM((2, SB, CH), jnp.int32),
            pltpu.VMEM((NB, CH, h), jnp.float32),
            pltpu.VMEM_SHARED((npad, h), jnp.float32),
            [pltpu.SemaphoreType.DMA] * NB,
            pltpu.SemaphoreType.DMA,
            pltpu.SemaphoreType.DMA,
        ],
    )
    def sc_scatter(xall, g3, d3, out, gidx, didx, rows, acc, gsem, sg, sd):
        c = lax.axis_index("c")
        s = lax.axis_index("s")
        w = c * NS + s

        def zbody(i, _):
            for k in range(h // 16):
                rows[0, i, pl.ds(k * 16, 16)] = jnp.zeros((16,), jnp.float32)
            return 0

        lax.fori_loop(0, CH, zbody, 0)
        nfull, rem = rows_per_sub // CH, rows_per_sub % CH
        for q in range(nfull):
            pltpu.sync_copy(
                rows.at[0], acc.at[pl.ds(s * rows_per_sub + q * CH, CH)])
        if rem:
            pltpu.sync_copy(
                rows.at[0, :rem], acc.at[pl.ds(s * rows_per_sub + nfull * CH, rem)])
        plsc.subcore_barrier()

        pltpu.sync_copy(g3.at[w, 0], gidx.at[0])
        pltpu.sync_copy(d3.at[w, 0], didx.at[0])
        pltpu.async_copy(g3.at[w, 1], gidx.at[1], sg)
        pltpu.async_copy(d3.at[w, 1], didx.at[1], sd)
        for b in range(NB):
            pltpu.async_copy(xall.at[gidx.at[0, b]], rows.at[b], gsem[b])

        def sb_body(sb, _):
            p = lax.rem(sb, 2)

            def quad_body(q, _):
                j = q * NB
                for b in range(NB):
                    pltpu.make_async_copy(
                        xall.at[gidx.at[p, j + b]], rows.at[b], gsem[b]).wait()
                    pltpu.sync_copy(
                        rows.at[b], acc.at[didx.at[p, j + b]], add=True)
                    pltpu.async_copy(
                        xall.at[gidx.at[p, j + NB + b]], rows.at[b], gsem[b])
                return 0

            lax.fori_loop(0, SB // NB - 1, quad_body, 0)

            for b in range(NB):
                pltpu.make_async_copy(
                    xall.at[gidx.at[p, SB - NB + b]], rows.at[b],
                    gsem[b]).wait()
                pltpu.sync_copy(
                    rows.at[b], acc.at[didx.at[p, SB - NB + b]], add=True)

            @pl.when(sb + 1 < nsb)
            def _refill():
                pltpu.make_async_copy(g3.at[w, 0], gidx.at[0], sg).wait()
                pltpu.make_async_copy(d3.at[w, 0], didx.at[0], sd).wait()
                for b in range(NB):
                    pltpu.async_copy(
                        xall.at[gidx.at[1 - p, b]], rows.at[b], gsem[b])

            @pl.when(sb + 2 < nsb)
            def _prefetch():
                pltpu.async_copy(g3.at[w, sb + 2], gidx.at[p], sg)
                pltpu.async_copy(d3.at[w, sb + 2], didx.at[p], sd)

            return 0

        lax.fori_loop(0, nsb, sb_body, 0)
        plsc.subcore_barrier()

        pltpu.sync_copy(
            acc.at[pl.ds(s * rows_per_sub, rows_per_sub)],
            out.at[c, pl.ds(s * rows_per_sub, rows_per_sub)],
        )

    return sc_scatter


def _gru_body(i_ref, p_ref, wir_ref, bir_ref, wiz_ref, biz_ref,
              win_ref, bin_ref, whr_ref, whz_ref, whn_ref, bhn_ref, out_ref):
    x = i_ref[...]
    h = p_ref[0] + p_ref[1]

    def mm(a, w_ref):
        return jnp.dot(a, w_ref[...], preferred_element_type=jnp.float32)

    r = jax.nn.sigmoid(mm(x, wir_ref) + bir_ref[...] + mm(h, whr_ref))
    z = jax.nn.sigmoid(mm(x, wiz_ref) + biz_ref[...] + mm(h, whz_ref))
    nn = jnp.tanh(mm(x, win_ref) + bin_ref[...] + r * (mm(h, whn_ref) + bhn_ref[...]))
    out_ref[...] = (1.0 - z) * nn + z * h


def _gru(emb, partials, w_ir, b_ir, w_iz, b_iz, w_in, b_in, w_hr, w_hz,
         w_hn, b_hn):
    n, h = emb.shape
    blk = 2000
    grid = n // blk
    row_spec = pl.BlockSpec((blk, h), lambda i: (i, 0))
    p_spec = pl.BlockSpec((2, blk, h), lambda i: (0, i, 0))
    w_spec = pl.BlockSpec((h, h), lambda i: (0, 0))
    b_spec = pl.BlockSpec((1, h), lambda i: (0, 0))
    return pl.pallas_call(
        _gru_body,
        grid=(grid,),
        in_specs=[row_spec, p_spec,
                  w_spec, b_spec, w_spec, b_spec, w_spec, b_spec,
                  w_spec, w_spec, w_spec, b_spec],
        out_specs=row_spec,
        out_shape=jax.ShapeDtypeStruct((n, h), jnp.float32),
    )(emb, partials,
      w_ir, b_ir.reshape(1, h), w_iz, b_iz.reshape(1, h),
      w_in, b_in.reshape(1, h),
      w_hr, w_hz, w_hn, b_hn.reshape(1, h))


def kernel(statement_embeddings, source_indices, dest_indices, edge_types,
           num_nodes, hidden_size, config, W_edge, b_edge, W_ir, b_ir,
           W_iz, b_iz, W_in, b_in, W_hr, W_hz, W_hn, b_hn):
    n, h = statement_embeddings.shape
    e = source_indices.shape[0]

    npad = ((n + 8 * NS - 1) // (8 * NS)) * (8 * NS)
    nw = NC * NS
    x_all = _edge_table(statement_embeddings, W_edge, b_edge)
    x_all = x_all.reshape(n * T, h)
    nsb = e // (nw * SB * CH)
    g4 = (source_indices * T + edge_types).reshape(nw, nsb, SB, CH)
    dest4 = dest_indices.reshape(nw, nsb, SB, CH)
    sc = _make_sc_scatter(npad, h, e)
    partials = sc(x_all, g4, dest4)

    return _gru(statement_embeddings, partials,
                W_ir, b_ir, W_iz, b_iz, W_in, b_in, W_hr, W_hz, W_hn, b_hn)
